# baseline (speedup 1.0000x reference)
import jax
import jax.numpy as jnp
from jax.experimental import pallas as pl
from jax.experimental.pallas import tpu as pltpu


def _roll_kernel(shift_ref, x_ref, o_ref):
    o_ref[...] = pltpu.roll(x_ref[...], shift_ref[0], axis=1)


def kernel(x, shift):
    b, f, t = x.shape
    s = jnp.reshape(shift.astype(jnp.int32) % f, (1,))
    bb = 2
    grid = (b // bb,)
    return pl.pallas_call(
        _roll_kernel,
        out_shape=jax.ShapeDtypeStruct((b, f, t), x.dtype),
        grid_spec=pltpu.PrefetchScalarGridSpec(
            num_scalar_prefetch=1,
            grid=grid,
            in_specs=[pl.BlockSpec((bb, f, t), lambda i, s: (i, 0, 0))],
            out_specs=pl.BlockSpec((bb, f, t), lambda i, s: (i, 0, 0)),
        ),
        compiler_params=pltpu.CompilerParams(
            dimension_semantics=("parallel",),
            vmem_limit_bytes=32 * 1024 * 1024,
        ),
    )(s, x)


# speedup vs baseline: 1.5344x; 1.5344x over previous
import jax
import jax.numpy as jnp
from jax.experimental import pallas as pl
from jax.experimental.pallas import tpu as pltpu


def _roll_kernel(shift_ref, x_ref, o_ref):
    o_ref[...] = pltpu.roll(x_ref[...], shift_ref[0], axis=1)


def kernel(x, shift):
    b, f, t = x.shape
    s = jnp.reshape(shift.astype(jnp.int32) % f, (1,))
    bb = 8
    grid = (b // bb,)
    return pl.pallas_call(
        _roll_kernel,
        out_shape=jax.ShapeDtypeStruct((b, f, t), x.dtype),
        grid_spec=pltpu.PrefetchScalarGridSpec(
            num_scalar_prefetch=1,
            grid=grid,
            in_specs=[pl.BlockSpec((bb, f, t), lambda i, s: (i, 0, 0))],
            out_specs=pl.BlockSpec((bb, f, t), lambda i, s: (i, 0, 0)),
        ),
        compiler_params=pltpu.CompilerParams(
            dimension_semantics=("parallel",),
            vmem_limit_bytes=32 * 1024 * 1024,
        ),
    )(s, x)


# speedup vs baseline: 1.5724x; 1.0247x over previous
import jax
import jax.numpy as jnp
from jax.experimental import pallas as pl
from jax.experimental.pallas import tpu as pltpu


def _roll_kernel(shift_ref, x_ref, o_ref):
    o_ref[...] = pltpu.roll(x_ref[...], shift_ref[0], axis=1)


def kernel(x, shift):
    b, f, t = x.shape
    s = jnp.reshape(shift.astype(jnp.int32) % f, (1,))
    bb = 16
    grid = (b // bb,)
    return pl.pallas_call(
        _roll_kernel,
        out_shape=jax.ShapeDtypeStruct((b, f, t), x.dtype),
        grid_spec=pltpu.PrefetchScalarGridSpec(
            num_scalar_prefetch=1,
            grid=grid,
            in_specs=[pl.BlockSpec((bb, f, t), lambda i, s: (i, 0, 0))],
            out_specs=pl.BlockSpec((bb, f, t), lambda i, s: (i, 0, 0)),
        ),
        compiler_params=pltpu.CompilerParams(
            dimension_semantics=("parallel",),
            vmem_limit_bytes=60 * 1024 * 1024,
        ),
    )(s, x)
